# baseline (speedup 1.0000x reference)
import functools

import jax
import jax.numpy as jnp
import numpy as np
from jax import lax
from jax.experimental import pallas as pl
from jax.experimental.pallas import tpu as pltpu
from jax.experimental.pallas import tpu_sc as plsc

def _hsum16(v):
    lanes = lax.iota(jnp.int32, 16)
    for m in (8, 4, 2, 1):
        idx = lanes ^ m
        v = v + v.at[idx].get(mode="promise_in_bounds", unique_indices=True)
    return v

N = 10000
DEG = 32
D = 128
NPAD = 10240
NW = 32
NPW = NPAD // NW
C = 4
NCHUNK = NPW // C
NBUF = 2
NK = D // 16


def _mm_body(x_ref, w_ref, o_ref):
    o_ref[...] = jnp.dot(x_ref[...], w_ref[...],
                         preferred_element_type=jnp.float32)


def _matmul(xpad, w):
    return pl.pallas_call(
        _mm_body,
        grid=(NPAD // 1024,),
        in_specs=[
            pl.BlockSpec((1024, D), lambda i: (i, 0)),
            pl.BlockSpec((D, D), lambda i: (0, 0)),
        ],
        out_specs=pl.BlockSpec((1024, D), lambda i: (i, 0)),
        out_shape=jax.ShapeDtypeStruct((NPAD, D), jnp.float32),
    )(xpad, w)


def _sc_body(xp, colp, av, out, idx_v, gbuf, obuf, ownb, avb, shared,
             gsem0, gsem1, osem0, osem1, wsem0, wsem1):
    sid = lax.axis_index("s")
    wid = sid * 2 + lax.axis_index("c")
    base = wid * NPW
    rps = NPAD // 16
    pltpu.sync_copy(xp.at[pl.ds(sid * rps, rps)],
                    shared.at[pl.ds(sid * rps, rps)])
    pltpu.sync_copy(colp.at[pl.ds(base * DEG, NPW * DEG)], idx_v)
    pltpu.sync_copy(av, avb)
    plsc.subcore_barrier()
    gsems = (gsem0, gsem1)
    osems = (osem0, osem1)
    wsems = (wsem0, wsem1)

    def gather_copy(g, b):
        idxsl = idx_v.at[pl.ds(g * (C * DEG), C * DEG)]
        return pltpu.make_async_copy(shared.at[idxsl], gbuf.at[b], gsems[b])

    def own_copy(g, b):
        return pltpu.make_async_copy(shared.at[pl.ds(base + g * C, C)],
                                     ownb.at[b], osems[b])

    def out_copy(g, b):
        return pltpu.make_async_copy(obuf.at[b],
                                     out.at[pl.ds(base + g * C, C)], wsems[b])

    for b in range(NBUF):
        gather_copy(b, b).start()
        own_copy(b, b).start()
    avv = avb[...]

    def chunk_body(g2, carry):
        for b in range(NBUF):
            g = g2 * NBUF + b
            own_copy(g, b).wait()
            gather_copy(g, b).wait()

            @pl.when(g2 > 0)
            def _():
                out_copy(g - NBUF, b).wait()

            @plsc.parallel_loop(0, C, 1, unroll=2)
            def node_body(n):
                e0 = n * DEG
                xs = [ownb[b, n, pl.ds(16 * k, 16)] * avv
                      for k in range(NK)]
                acc = [jnp.zeros((16,), jnp.float32)] * NK
                for j in range(DEG):
                    gv = [gbuf[b, e0 + j, pl.ds(16 * k, 16)]
                          for k in range(NK)]
                    p = [gv[k] * xs[k] for k in range(NK)]
                    s = (((p[0] + p[1]) + (p[2] + p[3]))
                         + ((p[4] + p[5]) + (p[6] + p[7])))
                    ef = _hsum16(s)
                    for k in range(NK):
                        acc[k] = acc[k] + ef * gv[k]
                for k in range(NK):
                    obuf[b, n, pl.ds(16 * k, 16)] = acc[k]

            out_copy(g, b).start()

            @pl.when(g + NBUF < NCHUNK)
            def _():
                gather_copy(g + NBUF, b).start()
                own_copy(g + NBUF, b).start()
        return carry

    lax.fori_loop(0, NCHUNK // NBUF, chunk_body, 0)
    for b in range(NBUF):
        out_copy(NCHUNK - NBUF + b, b).wait()


_sc_call = functools.partial(
    pl.kernel,
    out_type=jax.ShapeDtypeStruct((NPAD, D), jnp.float32),
    mesh=plsc.VectorSubcoreMesh(core_axis_name="c", subcore_axis_name="s"),
    scratch_types=[
        pltpu.VMEM((NPW * DEG,), jnp.int32),
        pltpu.VMEM((NBUF, C * DEG, D), jnp.float32),
        pltpu.VMEM((NBUF, C, D), jnp.float32),
        pltpu.VMEM((NBUF, C, D), jnp.float32),
        pltpu.VMEM((16,), jnp.float32),
        pltpu.VMEM_SHARED((NPAD, D), jnp.float32),
        pltpu.SemaphoreType.DMA,
        pltpu.SemaphoreType.DMA,
        pltpu.SemaphoreType.DMA,
        pltpu.SemaphoreType.DMA,
        pltpu.SemaphoreType.DMA,
        pltpu.SemaphoreType.DMA,
    ],
)(_sc_body)


def kernel(X, W, attention_w, row_pointers, column_index, blockPartition,
           edgeToColumn, edgeToRow, orign):
    xpad = jnp.zeros((NPAD, D), jnp.float32).at[:N].set(X)
    xp = _matmul(xpad, W)
    colp = jnp.zeros((NPAD * DEG,), jnp.int32).at[:N * DEG].set(column_index)
    av = jnp.broadcast_to(attention_w.reshape(-1)[:1], (16,)).astype(jnp.float32)
    out = _sc_call(xp, colp, av)
    return out[:N]


# speedup vs baseline: 1.5946x; 1.5946x over previous
import functools

import jax
import jax.numpy as jnp
import numpy as np
from jax import lax
from jax.experimental import pallas as pl
from jax.experimental.pallas import tpu as pltpu
from jax.experimental.pallas import tpu_sc as plsc

def _hsum16(v):
    lanes = lax.iota(jnp.int32, 16)
    for m in (8, 4, 2, 1):
        idx = lanes ^ m
        v = v + v.at[idx].get(mode="promise_in_bounds", unique_indices=True)
    return v

N = 10000
DEG = 32
D = 128
NPAD = 10240
NW = 32
NPW = NPAD // NW
C = 4
NCHUNK = NPW // C
NBUF = 2
NK = D // 16


def _mm_body(x_ref, w_ref, o_ref):
    o_ref[...] = jnp.dot(x_ref[...], w_ref[...],
                         preferred_element_type=jnp.float32)


def _matmul(xpad, w):
    return pl.pallas_call(
        _mm_body,
        grid=(NPAD // 1024,),
        in_specs=[
            pl.BlockSpec((1024, D), lambda i: (i, 0)),
            pl.BlockSpec((D, D), lambda i: (0, 0)),
        ],
        out_specs=pl.BlockSpec((1024, D), lambda i: (i, 0)),
        out_shape=jax.ShapeDtypeStruct((NPAD, D), jnp.float32),
    )(xpad, w)


def _sc_body(xp, colp, av, out, idx_v, gbuf, obuf, ownb, avb, shared,
             gsem0, gsem1, osem0, osem1, wsem0, wsem1):
    sid = lax.axis_index("s")
    wid = sid * 2 + lax.axis_index("c")
    base = wid * NPW
    rps = NPAD // 16
    pltpu.sync_copy(xp.at[pl.ds(sid * rps, rps)],
                    shared.at[pl.ds(sid * rps, rps)])
    pltpu.sync_copy(colp.at[pl.ds(base * DEG, NPW * DEG)], idx_v)
    pltpu.sync_copy(av, avb)
    plsc.subcore_barrier()
    gsems = (gsem0, gsem1)
    osems = (osem0, osem1)
    wsems = (wsem0, wsem1)

    def gather_copy(g, b):
        idxsl = idx_v.at[pl.ds(g * (C * DEG), C * DEG)]
        return pltpu.make_async_copy(shared.at[idxsl], gbuf.at[b], gsems[b])

    def own_copy(g, b):
        return pltpu.make_async_copy(shared.at[pl.ds(base + g * C, C)],
                                     ownb.at[b], osems[b])

    def out_copy(g, b):
        return pltpu.make_async_copy(obuf.at[b],
                                     out.at[pl.ds(base + g * C, C)], wsems[b])

    for b in range(NBUF):
        gather_copy(b, b).start()
        own_copy(b, b).start()
    avv = avb[...]

    def chunk_body(g2, carry):
        for b in range(NBUF):
            g = g2 * NBUF + b
            own_copy(g, b).wait()
            gather_copy(g, b).wait()

            @pl.when(g2 > 0)
            def _():
                out_copy(g - NBUF, b).wait()

            @plsc.parallel_loop(0, C, 1)
            def node_body(n):
                e0 = n * DEG
                xs = [ownb[b, n, pl.ds(16 * k, 16)] * avv
                      for k in range(NK)]
                acc = [jnp.zeros((16,), jnp.float32)] * NK
                for j in range(DEG):
                    gv = [gbuf[b, e0 + j, pl.ds(16 * k, 16)]
                          for k in range(NK)]
                    p = [gv[k] * xs[k] for k in range(NK)]
                    s = (((p[0] + p[1]) + (p[2] + p[3]))
                         + ((p[4] + p[5]) + (p[6] + p[7])))
                    ef = _hsum16(s)
                    for k in range(NK):
                        acc[k] = acc[k] + ef * gv[k]
                for k in range(NK):
                    obuf[b, n, pl.ds(16 * k, 16)] = acc[k]

            out_copy(g, b).start()

            @pl.when(g + NBUF < NCHUNK)
            def _():
                gather_copy(g + NBUF, b).start()
                own_copy(g + NBUF, b).start()
        return carry

    lax.fori_loop(0, NCHUNK // NBUF, chunk_body, 0)
    for b in range(NBUF):
        out_copy(NCHUNK - NBUF + b, b).wait()


_sc_call = functools.partial(
    pl.kernel,
    out_type=jax.ShapeDtypeStruct((NPAD, D), jnp.float32),
    mesh=plsc.VectorSubcoreMesh(core_axis_name="c", subcore_axis_name="s"),
    scratch_types=[
        pltpu.VMEM((NPW * DEG,), jnp.int32),
        pltpu.VMEM((NBUF, C * DEG, D), jnp.float32),
        pltpu.VMEM((NBUF, C, D), jnp.float32),
        pltpu.VMEM((NBUF, C, D), jnp.float32),
        pltpu.VMEM((16,), jnp.float32),
        pltpu.VMEM_SHARED((NPAD, D), jnp.float32),
        pltpu.SemaphoreType.DMA,
        pltpu.SemaphoreType.DMA,
        pltpu.SemaphoreType.DMA,
        pltpu.SemaphoreType.DMA,
        pltpu.SemaphoreType.DMA,
        pltpu.SemaphoreType.DMA,
    ],
)(_sc_body)


def kernel(X, W, attention_w, row_pointers, column_index, blockPartition,
           edgeToColumn, edgeToRow, orign):
    xpad = jnp.zeros((NPAD, D), jnp.float32).at[:N].set(X)
    xp = _matmul(xpad, W)
    colp = jnp.zeros((NPAD * DEG,), jnp.int32).at[:N * DEG].set(column_index)
    av = jnp.broadcast_to(attention_w.reshape(-1)[:1], (16,)).astype(jnp.float32)
    out = _sc_call(xp, colp, av)
    return out[:N]


# speedup vs baseline: 1.6498x; 1.0346x over previous
import functools

import jax
import jax.numpy as jnp
import numpy as np
from jax import lax
from jax.experimental import pallas as pl
from jax.experimental.pallas import tpu as pltpu
from jax.experimental.pallas import tpu_sc as plsc

def _hsum16(v):
    lanes = lax.iota(jnp.int32, 16)
    for m in (8, 4, 2, 1):
        idx = lanes ^ m
        v = v + v.at[idx].get(mode="promise_in_bounds", unique_indices=True)
    return v

N = 10000
DEG = 32
D = 128
NPAD = 10240
NW = 32
NPW = NPAD // NW
C = 4
NCHUNK = NPW // C
NBUF = 2
NK = D // 16


def _mm_body(x_ref, w_ref, o_ref):
    o_ref[...] = jnp.dot(x_ref[...], w_ref[...],
                         preferred_element_type=jnp.float32)


def _matmul(x, w):
    return pl.pallas_call(
        _mm_body,
        grid=(NPAD // 1024,),
        in_specs=[
            pl.BlockSpec((1024, D), lambda i: (i, 0)),
            pl.BlockSpec((D, D), lambda i: (0, 0)),
        ],
        out_specs=pl.BlockSpec((1024, D), lambda i: (i, 0)),
        out_shape=jax.ShapeDtypeStruct((NPAD, D), jnp.float32),
    )(x, w)


def _sc_body(xp, colp, av, out, idx_v, gbuf, obuf, ownb, avb, shared,
             gsem0, gsem1, osem0, osem1, wsem0, wsem1):
    sid = lax.axis_index("s")
    wid = sid * 2 + lax.axis_index("c")
    base = wid * NPW
    rps = NPAD // 16
    pltpu.sync_copy(xp.at[pl.ds(sid * rps, rps)],
                    shared.at[pl.ds(sid * rps, rps)])
    pltpu.sync_copy(colp.at[pl.ds(base * DEG, NPW * DEG)], idx_v)
    pltpu.sync_copy(av, avb)
    plsc.subcore_barrier()
    gsems = (gsem0, gsem1)
    osems = (osem0, osem1)
    wsems = (wsem0, wsem1)

    def gather_copy(g, b):
        idxsl = idx_v.at[pl.ds(g * (C * DEG), C * DEG)]
        return pltpu.make_async_copy(shared.at[idxsl], gbuf.at[b], gsems[b])

    def own_copy(g, b):
        return pltpu.make_async_copy(shared.at[pl.ds(base + g * C, C)],
                                     ownb.at[b], osems[b])

    def out_copy(g, b):
        return pltpu.make_async_copy(obuf.at[b],
                                     out.at[pl.ds(base + g * C, C)], wsems[b])

    for b in range(NBUF):
        gather_copy(b, b).start()
        own_copy(b, b).start()
    avv = avb[...]

    def chunk_body(g2, carry):
        for b in range(NBUF):
            g = g2 * NBUF + b
            own_copy(g, b).wait()
            gather_copy(g, b).wait()

            @pl.when(g2 > 0)
            def _():
                out_copy(g - NBUF, b).wait()

            @plsc.parallel_loop(0, C, 1)
            def node_body(n):
                e0 = n * DEG
                xs = [ownb[b, n, pl.ds(16 * k, 16)] * avv
                      for k in range(NK)]
                acc = [jnp.zeros((16,), jnp.float32)] * NK
                for j in range(DEG):
                    gv = [gbuf[b, e0 + j, pl.ds(16 * k, 16)]
                          for k in range(NK)]
                    p = [gv[k] * xs[k] for k in range(NK)]
                    s = (((p[0] + p[1]) + (p[2] + p[3]))
                         + ((p[4] + p[5]) + (p[6] + p[7])))
                    ef = _hsum16(s)
                    for k in range(NK):
                        acc[k] = acc[k] + ef * gv[k]
                for k in range(NK):
                    obuf[b, n, pl.ds(16 * k, 16)] = acc[k]

            out_copy(g, b).start()

            @pl.when(g + NBUF < NCHUNK)
            def _():
                gather_copy(g + NBUF, b).start()
                own_copy(g + NBUF, b).start()
        return carry

    lax.fori_loop(0, NCHUNK // NBUF, chunk_body, 0)
    for b in range(NBUF):
        out_copy(NCHUNK - NBUF + b, b).wait()


_sc_call = functools.partial(
    pl.kernel,
    out_type=jax.ShapeDtypeStruct((NPAD, D), jnp.float32),
    mesh=plsc.VectorSubcoreMesh(core_axis_name="c", subcore_axis_name="s"),
    scratch_types=[
        pltpu.VMEM((NPW * DEG,), jnp.int32),
        pltpu.VMEM((NBUF, C * DEG, D), jnp.float32),
        pltpu.VMEM((NBUF, C, D), jnp.float32),
        pltpu.VMEM((NBUF, C, D), jnp.float32),
        pltpu.VMEM((16,), jnp.float32),
        pltpu.VMEM_SHARED((NPAD, D), jnp.float32),
        pltpu.SemaphoreType.DMA,
        pltpu.SemaphoreType.DMA,
        pltpu.SemaphoreType.DMA,
        pltpu.SemaphoreType.DMA,
        pltpu.SemaphoreType.DMA,
        pltpu.SemaphoreType.DMA,
    ],
)(_sc_body)


def kernel(X, W, attention_w, row_pointers, column_index, blockPartition,
           edgeToColumn, edgeToRow, orign):
    xp = _matmul(X, W)
    colp = jnp.zeros((NPAD * DEG,), jnp.int32).at[:N * DEG].set(column_index)
    av = jnp.broadcast_to(attention_w.reshape(-1)[:1], (16,)).astype(jnp.float32)
    out = _sc_call(xp, colp, av)
    return out[:N]


# speedup vs baseline: 1.6673x; 1.0106x over previous
import functools

import jax
import jax.numpy as jnp
import numpy as np
from jax import lax
from jax.experimental import pallas as pl
from jax.experimental.pallas import tpu as pltpu
from jax.experimental.pallas import tpu_sc as plsc

def _hsum16(v):
    lanes = lax.iota(jnp.int32, 16)
    for m in (8, 4, 2, 1):
        idx = lanes ^ m
        v = v + v.at[idx].get(mode="promise_in_bounds", unique_indices=True)
    return v

N = 10000
DEG = 32
D = 128
NPAD = 10240
NW = 32
NPW = NPAD // NW
C = 4
NCHUNK = NPW // C
NBUF = 2
NK = D // 16


def _mm_body(x_ref, w_ref, o_ref):
    o_ref[...] = jnp.dot(x_ref[...], w_ref[...],
                         preferred_element_type=jnp.float32)


def _matmul(x, w):
    return pl.pallas_call(
        _mm_body,
        grid=(NPAD // 1024,),
        in_specs=[
            pl.BlockSpec((1024, D), lambda i: (i, 0)),
            pl.BlockSpec((D, D), lambda i: (0, 0)),
        ],
        out_specs=pl.BlockSpec((1024, D), lambda i: (i, 0)),
        out_shape=jax.ShapeDtypeStruct((NPAD, D), jnp.float32),
    )(x, w)


def _sc_body(xp, colp, av, out, idx_v, gbuf, obuf, ownb, avb, shared,
             gsem0, gsem1, osem0, osem1, wsem0, wsem1):
    sid = lax.axis_index("s")
    wid = sid * 2 + lax.axis_index("c")
    base = wid * NPW
    rps = NPAD // 16

    def stage_copies():
        yield pltpu.make_async_copy(xp.at[pl.ds(sid * rps, rps)],
                                    shared.at[pl.ds(sid * rps, rps)], wsem0)
        yield pltpu.make_async_copy(colp.at[pl.ds(base * DEG, NPW * DEG)],
                                    idx_v, wsem1)
        yield pltpu.make_async_copy(av, avb, osem0)

    for cp in stage_copies():
        cp.start()
    for cp in stage_copies():
        cp.wait()
    plsc.subcore_barrier()
    gsems = (gsem0, gsem1)
    osems = (osem0, osem1)
    wsems = (wsem0, wsem1)

    def gather_copy(g, b):
        idxsl = idx_v.at[pl.ds(g * (C * DEG), C * DEG)]
        return pltpu.make_async_copy(shared.at[idxsl], gbuf.at[b], gsems[b])

    def own_copy(g, b):
        return pltpu.make_async_copy(shared.at[pl.ds(base + g * C, C)],
                                     ownb.at[b], osems[b])

    def out_copy(g, b):
        return pltpu.make_async_copy(obuf.at[b],
                                     out.at[pl.ds(base + g * C, C)], wsems[b])

    for b in range(NBUF):
        gather_copy(b, b).start()
        own_copy(b, b).start()
    avv = avb[...]

    def chunk_body(g2, carry):
        for b in range(NBUF):
            g = g2 * NBUF + b
            own_copy(g, b).wait()
            gather_copy(g, b).wait()

            @pl.when(g2 > 0)
            def _():
                out_copy(g - NBUF, b).wait()

            @plsc.parallel_loop(0, C, 1)
            def node_body(n):
                e0 = n * DEG
                xs = [ownb[b, n, pl.ds(16 * k, 16)] * avv
                      for k in range(NK)]
                acc = [jnp.zeros((16,), jnp.float32)] * NK
                for j in range(DEG):
                    gv = [gbuf[b, e0 + j, pl.ds(16 * k, 16)]
                          for k in range(NK)]
                    p = [gv[k] * xs[k] for k in range(NK)]
                    s = (((p[0] + p[1]) + (p[2] + p[3]))
                         + ((p[4] + p[5]) + (p[6] + p[7])))
                    ef = _hsum16(s)
                    for k in range(NK):
                        acc[k] = acc[k] + ef * gv[k]
                for k in range(NK):
                    obuf[b, n, pl.ds(16 * k, 16)] = acc[k]

            out_copy(g, b).start()

            @pl.when(g + NBUF < NCHUNK)
            def _():
                gather_copy(g + NBUF, b).start()
                own_copy(g + NBUF, b).start()
        return carry

    lax.fori_loop(0, NCHUNK // NBUF, chunk_body, 0)
    for b in range(NBUF):
        out_copy(NCHUNK - NBUF + b, b).wait()


_sc_call = functools.partial(
    pl.kernel,
    out_type=jax.ShapeDtypeStruct((NPAD, D), jnp.float32),
    mesh=plsc.VectorSubcoreMesh(core_axis_name="c", subcore_axis_name="s"),
    scratch_types=[
        pltpu.VMEM((NPW * DEG,), jnp.int32),
        pltpu.VMEM((NBUF, C * DEG, D), jnp.float32),
        pltpu.VMEM((NBUF, C, D), jnp.float32),
        pltpu.VMEM((NBUF, C, D), jnp.float32),
        pltpu.VMEM((16,), jnp.float32),
        pltpu.VMEM_SHARED((NPAD, D), jnp.float32),
        pltpu.SemaphoreType.DMA,
        pltpu.SemaphoreType.DMA,
        pltpu.SemaphoreType.DMA,
        pltpu.SemaphoreType.DMA,
        pltpu.SemaphoreType.DMA,
        pltpu.SemaphoreType.DMA,
    ],
)(_sc_body)


def kernel(X, W, attention_w, row_pointers, column_index, blockPartition,
           edgeToColumn, edgeToRow, orign):
    xp = _matmul(X, W)
    colp = jnp.zeros((NPAD * DEG,), jnp.int32).at[:N * DEG].set(column_index)
    av = jnp.broadcast_to(attention_w.reshape(-1)[:1], (16,)).astype(jnp.float32)
    out = _sc_call(xp, colp, av)
    return out[:N]
